# baseline (speedup 1.0000x reference)
import jax
import jax.numpy as jnp
from jax.experimental import pallas as pl
from jax.experimental.pallas import tpu as pltpu

_N_E = 8192
_E_DIM = 256
_BETA = 0.25
_T = 256
_SLAB1 = 2736
_SLAB2 = 5472


def _vq_kernel(zt_ref, se_ref, emb_ref, loss_ref, zq_ref, perp_ref, enc_ref,
               idx_ref, counts_sc, loss_sc):
    i = pl.program_id(0)
    n_tok = pl.num_programs(0) * _T

    zt = zt_ref[...]
    sz = jnp.sum(zt * zt, axis=1, keepdims=True)
    se = se_ref[...]
    emb = emb_ref[...]

    s = jax.lax.dot_general(zt, emb, (((1,), (1,)), ((), ())),
                            preferred_element_type=jnp.float32)
    d = (sz + se) - 2.0 * s

    io = jax.lax.broadcasted_iota(jnp.int32, d.shape, 1)
    inf = jnp.float32(jnp.inf)
    m0 = jnp.min(jnp.where(io < _SLAB1, d, inf), axis=1, keepdims=True)
    m1 = jnp.min(jnp.where((io >= _SLAB1) & (io < _SLAB2), d, inf),
                 axis=1, keepdims=True)
    m2 = jnp.min(jnp.where(io >= _SLAB2, d, inf), axis=1, keepdims=True)
    q0 = m0.astype(jnp.bfloat16).astype(jnp.float32)
    t1 = m1 < q0
    v1 = jnp.where(t1, m1, q0)
    a1 = jnp.where(t1, jnp.int32(1), jnp.int32(0))
    q1 = v1.astype(jnp.bfloat16).astype(jnp.float32)
    t2 = m2 < q1
    a2 = jnp.where(t2, jnp.int32(2), a1)
    v_sel = jnp.where(t2, m2, jnp.where(t1, m1, m0))
    slab_id = ((io >= _SLAB1).astype(jnp.int32)
               + (io >= _SLAB2).astype(jnp.int32))
    hit = (d == v_sel) & (slab_id == a2)
    idx = jnp.min(jnp.where(hit, io, jnp.int32(_N_E)), axis=1)
    idx_ref[...] = idx[:, None]

    onehot = (io == idx[:, None]).astype(jnp.float32)
    enc_ref[...] = onehot

    zq = jax.lax.dot_general(onehot, emb, (((1,), (0,)), ((), ())),
                             preferred_element_type=jnp.float32)
    zq_ref[...] = zt + (zq - zt)

    @pl.when(i == 0)
    def _init():
        counts_sc[...] = jnp.zeros_like(counts_sc)
        loss_sc[...] = jnp.zeros_like(loss_sc)

    counts_sc[...] += jnp.sum(onehot, axis=0, keepdims=True)
    loss_sc[...] += jnp.sum(v_sel, keepdims=True)

    @pl.when(i == pl.num_programs(0) - 1)
    def _finalize():
        loss_ref[...] = (1.0 + _BETA) * loss_sc[...] / (n_tok * _E_DIM)
        e_mean = counts_sc[...] / n_tok
        ent = jnp.sum(e_mean * jnp.log(e_mean + 1e-10), keepdims=True)
        perp_ref[...] = jnp.exp(-ent)


def kernel(z, emb):
    B, C, H, W = z.shape
    zp = jnp.transpose(z, (0, 2, 3, 1))
    z_flat = zp.reshape(-1, _E_DIM)
    n_tok = z_flat.shape[0]
    se = jnp.sum(emb ** 2, axis=1)[None, :]

    grid = (n_tok // _T,)
    loss, zq_flat, perp, enc, idx = pl.pallas_call(
        _vq_kernel,
        grid=grid,
        in_specs=[
            pl.BlockSpec((_T, _E_DIM), lambda i: (i, 0)),
            pl.BlockSpec((1, _N_E), lambda i: (0, 0)),
            pl.BlockSpec((_N_E, _E_DIM), lambda i: (0, 0)),
        ],
        out_specs=[
            pl.BlockSpec((1, 1), lambda i: (0, 0)),
            pl.BlockSpec((_T, _E_DIM), lambda i: (i, 0)),
            pl.BlockSpec((1, 1), lambda i: (0, 0)),
            pl.BlockSpec((_T, _N_E), lambda i: (i, 0)),
            pl.BlockSpec((_T, 1), lambda i: (i, 0)),
        ],
        out_shape=[
            jax.ShapeDtypeStruct((1, 1), jnp.float32),
            jax.ShapeDtypeStruct((n_tok, _E_DIM), jnp.float32),
            jax.ShapeDtypeStruct((1, 1), jnp.float32),
            jax.ShapeDtypeStruct((n_tok, _N_E), jnp.float32),
            jax.ShapeDtypeStruct((n_tok, 1), jnp.int32),
        ],
        scratch_shapes=[
            pltpu.VMEM((1, _N_E), jnp.float32),
            pltpu.VMEM((1, 1), jnp.float32),
        ],
    )(z_flat, se, emb)

    z_q = jnp.transpose(zq_flat.reshape(B, H, W, C), (0, 3, 1, 2))
    return (loss[0, 0], z_q, perp[0, 0], enc, idx)


# speedup vs baseline: 1.3694x; 1.3694x over previous
import functools

import jax
import jax.numpy as jnp
from jax import lax
from jax.experimental import pallas as pl
from jax.experimental.pallas import tpu as pltpu
from jax.experimental.pallas import tpu_sc as plsc

_N_E = 8192
_E_DIM = 256
_BETA = 0.25
_T = 256
_SLAB1 = 2736
_SLAB2 = 5472


def _vq_kernel(zt_ref, se_ref, emb_ref, loss_ref, perp_ref, enc_ref,
               idx_ref, counts_sc, loss_sc):
    i = pl.program_id(0)
    n_tok = pl.num_programs(0) * _T

    zt = zt_ref[...]
    sz = jnp.sum(zt * zt, axis=1, keepdims=True)
    se = se_ref[...]
    emb = emb_ref[...]

    s = jax.lax.dot_general(zt, emb, (((1,), (1,)), ((), ())),
                            preferred_element_type=jnp.float32)
    d = (sz + se) - 2.0 * s

    io = jax.lax.broadcasted_iota(jnp.int32, d.shape, 1)
    inf = jnp.float32(jnp.inf)
    m0 = jnp.min(jnp.where(io < _SLAB1, d, inf), axis=1, keepdims=True)
    m1 = jnp.min(jnp.where((io >= _SLAB1) & (io < _SLAB2), d, inf),
                 axis=1, keepdims=True)
    m2 = jnp.min(jnp.where(io >= _SLAB2, d, inf), axis=1, keepdims=True)
    q0 = m0.astype(jnp.bfloat16).astype(jnp.float32)
    t1 = m1 < q0
    v1 = jnp.where(t1, m1, q0)
    a1 = jnp.where(t1, jnp.int32(1), jnp.int32(0))
    q1 = v1.astype(jnp.bfloat16).astype(jnp.float32)
    t2 = m2 < q1
    a2 = jnp.where(t2, jnp.int32(2), a1)
    v_sel = jnp.where(t2, m2, jnp.where(t1, m1, m0))
    slab_id = ((io >= _SLAB1).astype(jnp.int32)
               + (io >= _SLAB2).astype(jnp.int32))
    hit = (d == v_sel) & (slab_id == a2)
    idx = jnp.min(jnp.where(hit, io, jnp.int32(_N_E)), axis=1)
    idx_ref[...] = idx[:, None]

    onehot = (io == idx[:, None]).astype(jnp.float32)
    enc_ref[...] = onehot

    @pl.when(i == 0)
    def _init():
        counts_sc[...] = jnp.zeros_like(counts_sc)
        loss_sc[...] = jnp.zeros_like(loss_sc)

    counts_sc[...] += jnp.sum(onehot, axis=0, keepdims=True)
    loss_sc[...] += jnp.sum(v_sel, keepdims=True)

    @pl.when(i == pl.num_programs(0) - 1)
    def _finalize():
        loss_ref[...] = (1.0 + _BETA) * loss_sc[...] / (n_tok * _E_DIM)
        e_mean = counts_sc[...] / n_tok
        ent = jnp.sum(e_mean * jnp.log(e_mean + 1e-10), keepdims=True)
        perp_ref[...] = jnp.exp(-ent)


def _make_sc_gather(n_tok):
    info = plsc.get_sparse_core_info()
    nw = info.num_cores * info.num_subcores
    b_per_w = n_tok // nw
    chunk = min(b_per_w, 256)
    n_chunks = b_per_w // chunk
    mesh = plsc.VectorSubcoreMesh(core_axis_name="c", subcore_axis_name="s")

    @functools.partial(
        pl.kernel, mesh=mesh,
        out_type=jax.ShapeDtypeStruct((n_tok, _E_DIM), jnp.float32),
        scratch_types=[
            pltpu.VMEM((chunk,), jnp.int32),
            pltpu.VMEM((chunk, _E_DIM), jnp.float32),
            pltpu.SemaphoreType.DMA,
        ],
    )
    def gather_k(table_hbm, idx_hbm, out_hbm, idx_v, rows_v, sem):
        wid = lax.axis_index("s") * info.num_cores + lax.axis_index("c")
        for c in range(n_chunks):
            base = wid * b_per_w + c * chunk
            pltpu.sync_copy(idx_hbm.at[pl.ds(base, chunk)], idx_v)
            pltpu.async_copy(table_hbm.at[idx_v], rows_v, sem).wait()
            pltpu.sync_copy(rows_v, out_hbm.at[pl.ds(base, chunk)])

    return gather_k


def kernel(z, emb):
    B, C, H, W = z.shape
    zp = jnp.transpose(z, (0, 2, 3, 1))
    z_flat = zp.reshape(-1, _E_DIM)
    n_tok = z_flat.shape[0]
    se = jnp.sum(emb ** 2, axis=1)[None, :]

    grid = (n_tok // _T,)
    loss, perp, enc, idx = pl.pallas_call(
        _vq_kernel,
        grid=grid,
        in_specs=[
            pl.BlockSpec((_T, _E_DIM), lambda i: (i, 0)),
            pl.BlockSpec((1, _N_E), lambda i: (0, 0)),
            pl.BlockSpec((_N_E, _E_DIM), lambda i: (0, 0)),
        ],
        out_specs=[
            pl.BlockSpec((1, 1), lambda i: (0, 0)),
            pl.BlockSpec((1, 1), lambda i: (0, 0)),
            pl.BlockSpec((_T, _N_E), lambda i: (i, 0)),
            pl.BlockSpec((_T, 1), lambda i: (i, 0)),
        ],
        out_shape=[
            jax.ShapeDtypeStruct((1, 1), jnp.float32),
            jax.ShapeDtypeStruct((1, 1), jnp.float32),
            jax.ShapeDtypeStruct((n_tok, _N_E), jnp.float32),
            jax.ShapeDtypeStruct((n_tok, 1), jnp.int32),
        ],
        scratch_shapes=[
            pltpu.VMEM((1, _N_E), jnp.float32),
            pltpu.VMEM((1, 1), jnp.float32),
        ],
    )(z_flat, se, emb)

    zq_flat = _make_sc_gather(n_tok)(emb, idx.reshape(-1))
    z_q = jnp.transpose(zq_flat.reshape(B, H, W, C), (0, 3, 1, 2))
    return (loss[0, 0], z_q, perp[0, 0], enc, idx)
